# baseline (speedup 1.0000x reference)
import jax
import jax.numpy as jnp
from jax import lax
from jax.experimental import pallas as pl
from jax.experimental.pallas import tpu as pltpu
from jax.experimental.pallas import tpu_sc as plsc

N_NODES = 10000
NP = 10240
NFEAT = 128
NCLASS = 40
N_EDGES = 320000
K = 128
CHUNKS = 80
EPT = K * CHUNKS
NTILES = 32
EP = EPT * NTILES
RPT = NP // 16
BR = 1024


def _mesh():
    return plsc.VectorSubcoreMesh(core_axis_name="c", subcore_axis_name="s")


def _deg_body(rc_hbm, out_hbm, c0, c1, c2, c3, ones, dz, acc,
              si0, si1, si2, si3, ss0, ss1, ss2, ss3):
    cb = (c0, c1, c2, c3)
    si = (si0, si1, si2, si3)
    ss = (ss0, ss1, ss2, ss3)
    cid = lax.axis_index("c")
    sid = lax.axis_index("s")
    tid = cid * 16 + sid

    for j in range(8):
        ones[pl.ds(j * 16, 16)] = jnp.full((16,), 1.0, jnp.float32)

    def zstep(j, _):
        dz[pl.ds(j * 16, 16)] = jnp.zeros((16,), jnp.float32)
        return 0
    lax.fori_loop(0, RPT // 16, zstep, 0)
    pltpu.sync_copy(dz, acc.at[pl.ds(sid * RPT, RPT)])
    plsc.subcore_barrier()

    cbase = tid * CHUNKS

    def iload(g, j):
        pltpu.async_copy(rc_hbm.at[cbase + g, 1], cb[j], si[j])

    def iwait(j):
        pltpu.make_async_copy(rc_hbm.at[cbase, 1], cb[j], si[j]).wait()

    def sstart(j):
        pltpu.async_copy(ones, acc.at[cb[j]], ss[j], add=True)

    def swait(j):
        pltpu.make_async_copy(ones, acc.at[cb[j]], ss[j]).wait()

    iload(0, 0)

    def body(m, _):
        for b in range(4):
            g = 4 * m + b
            if b >= 3:
                swait((b - 3) % 4)
            else:
                @pl.when(m >= 1)
                def _(b=b):
                    swait((b + 1) % 4)
            if b < 3:
                iload(g + 1, b + 1)
            else:
                @pl.when(m < CHUNKS // 4 - 1)
                def _(g=g):
                    iload(g + 1, 0)
            iwait(b)
            sstart(b)
        return 0
    lax.fori_loop(0, CHUNKS // 4, body, 0)
    for j in (1, 2, 3):
        swait(j)

    plsc.subcore_barrier()
    pltpu.sync_copy(acc.at[pl.ds(sid * RPT, RPT)],
                    out_hbm.at[cid, pl.ds(sid * RPT, RPT)])


def _deg_call(rc):
    f = pl.kernel(
        _deg_body,
        out_type=jax.ShapeDtypeStruct((2, NP), jnp.float32),
        mesh=_mesh(),
        scratch_types=[
            pltpu.VMEM((K,), jnp.int32),
            pltpu.VMEM((K,), jnp.int32),
            pltpu.VMEM((K,), jnp.int32),
            pltpu.VMEM((K,), jnp.int32),
            pltpu.VMEM((K,), jnp.float32),
            pltpu.VMEM((RPT,), jnp.float32),
            pltpu.VMEM_SHARED((NP,), jnp.float32),
        ] + [pltpu.SemaphoreType.DMA] * 8,
    )
    return f(rc)


def _agg_body(hp_hbm, rc_hbm, out_hbm,
              rc0, rc1, rc2, rc3, rc4, rc5, rc6, rc7,
              rows0, rows1, acc,
              si0, si1, si2, si3, si4, si5, si6, si7,
              sg0, sg1, ss0, ss1):
    rc = (rc0, rc1, rc2, rc3, rc4, rc5, rc6, rc7)
    rows = (rows0, rows1)
    si = (si0, si1, si2, si3, si4, si5, si6, si7)
    sg = (sg0, sg1)
    ss = (ss0, ss1)
    cid = lax.axis_index("c")
    sid = lax.axis_index("s")
    tid = cid * 16 + sid

    cbase = tid * CHUNKS

    def iload(g, j):
        pltpu.async_copy(rc_hbm.at[cbase + g], rc[j], si[j])

    def iwait(j):
        pltpu.make_async_copy(rc_hbm.at[cbase], rc[j], si[j]).wait()

    iload(0, 0)
    iload(1, 1)

    def zstep(r, _):
        for j in range(8):
            rows0[r, pl.ds(j * 16, 16)] = jnp.zeros((16,), jnp.float32)
        return 0
    lax.fori_loop(0, K, zstep, 0)
    for k in range(RPT // K):
        pltpu.async_copy(rows0, acc.at[pl.ds(sid * RPT + k * K, K)], ss0)
    for k in range(RPT // K):
        pltpu.make_async_copy(rows0, acc.at[pl.ds(sid * RPT + k * K, K)],
                              ss0).wait()
    plsc.subcore_barrier()

    def gstart(jc, jr):
        pltpu.async_copy(hp_hbm.at[rc[jc].at[0]], rows[jr], sg[jr])

    def gwait(jr):
        pltpu.make_async_copy(hp_hbm.at[rc[0].at[0]], rows[jr], sg[jr]).wait()

    def sstart(jc, jr):
        pltpu.async_copy(rows[jr], acc.at[rc[jc].at[1]], ss[jr], add=True)

    def swait(jr):
        pltpu.make_async_copy(rows[0], acc.at[rc[0].at[1]], ss[jr]).wait()

    def body(m, _):
        for b in range(8):
            g_s = 8 * m + b
            if b >= 2:
                swait(b % 2)
            else:
                @pl.when(m >= 1)
                def _(b=b):
                    swait(b % 2)
            if b < 6:
                iload(g_s + 2, (b + 2) % 8)
            else:
                @pl.when(m < CHUNKS // 8 - 1)
                def _(g_s=g_s, b=b):
                    iload(g_s + 2, (b + 2) % 8)
            iwait(b)
            gstart(b, b % 2)
            if b >= 1:
                gwait((b - 1) % 2)
                sstart((b - 1) % 8, (b - 1) % 2)
            else:
                @pl.when(m >= 1)
                def _():
                    gwait(1)
                    sstart(7, 1)
        return 0
    lax.fori_loop(0, CHUNKS // 8, body, 0)
    gwait(1)
    sstart(7, 1)
    for j in range(2):
        swait(j)

    plsc.subcore_barrier()
    pltpu.sync_copy(acc.at[pl.ds(sid * RPT, RPT)],
                    out_hbm.at[cid, pl.ds(sid * RPT, RPT)])


def _agg_call(hp, rc):
    f = pl.kernel(
        _agg_body,
        out_type=jax.ShapeDtypeStruct((2, NP, NFEAT), jnp.float32),
        mesh=_mesh(),
        scratch_types=[pltpu.VMEM((2, K), jnp.int32)] * 8
        + [pltpu.VMEM((K, NFEAT), jnp.float32)] * 2
        + [pltpu.VMEM_SHARED((NP, NFEAT), jnp.float32)]
        + [pltpu.SemaphoreType.DMA] * 12,
    )
    return f(hp, rc)


def _s_of(dp_ref):
    d = dp_ref[0, :] + dp_ref[1, :] + 1.0
    return lax.rsqrt(d)


def _tc1a_body(x_ref, wpre_ref, bpre_ref, w1_ref, t1_ref):
    h0 = jnp.dot(x_ref[:], wpre_ref[:], preferred_element_type=jnp.float32)
    h0 = h0 + bpre_ref[:][None, :]
    t1_ref[:] = jnp.dot(h0, w1_ref[:], preferred_element_type=jnp.float32)


def _tc1b_body(t1_ref, dp_ref, hp1_ref, s_ref):
    s = _s_of(dp_ref)
    s_ref[:] = s[:, None]
    hp1_ref[:] = t1_ref[:] * s[:, None]


def _tc2_body(agg_ref, hp_ref, s_ref, b_ref, w2_ref, hp2_ref):
    s = s_ref[:]
    pre = s * (agg_ref[0] + agg_ref[1] + hp_ref[:]) + b_ref[:][None, :]
    h1 = jnp.maximum(pre, 0.0)
    t2 = jnp.dot(h1, w2_ref[:], preferred_element_type=jnp.float32)
    hp2_ref[:] = t2 * s


def _tc3_body(agg_ref, hp_ref, s_ref, b_ref, wpost_ref, bpost_ref, out_ref):
    s = s_ref[:]
    pre = s * (agg_ref[0] + agg_ref[1] + hp_ref[:]) + b_ref[:][None, :]
    h2 = jnp.maximum(pre, 0.0)
    logits = jnp.dot(h2, wpost_ref[:], preferred_element_type=jnp.float32)
    logits = logits + bpost_ref[:][None, :]
    m = jnp.max(logits, axis=1, keepdims=True)
    lse = jnp.log(jnp.sum(jnp.exp(logits - m), axis=1, keepdims=True)) + m
    out_ref[:] = logits - lse


def _row_spec(feat):
    return pl.BlockSpec((BR, feat), lambda i: (i, 0))


def _full_spec(shape):
    return pl.BlockSpec(shape, lambda i: tuple(0 for _ in shape))


_dp_spec = pl.BlockSpec((2, BR), lambda i: (0, i))
_agg_spec = pl.BlockSpec((2, BR, NFEAT), lambda i: (0, i, 0))
_grid = (NP // BR,)


def _tc1a(x_p, W_pre, b_pre, W1):
    return pl.pallas_call(
        _tc1a_body,
        grid=_grid,
        in_specs=[_row_spec(NFEAT), _full_spec((NFEAT, NFEAT)),
                  _full_spec((NFEAT,)), _full_spec((NFEAT, NFEAT))],
        out_specs=_row_spec(NFEAT),
        out_shape=jax.ShapeDtypeStruct((NP, NFEAT), jnp.float32),
    )(x_p, W_pre, b_pre, W1)


def _tc1b(t1, dp):
    return pl.pallas_call(
        _tc1b_body,
        grid=_grid,
        in_specs=[_row_spec(NFEAT), _dp_spec],
        out_specs=[_row_spec(NFEAT), _row_spec(1)],
        out_shape=[jax.ShapeDtypeStruct((NP, NFEAT), jnp.float32),
                   jax.ShapeDtypeStruct((NP, 1), jnp.float32)],
    )(t1, dp)


def _tc2(agg, hp, s2d, b1, W2):
    return pl.pallas_call(
        _tc2_body,
        grid=_grid,
        in_specs=[_agg_spec, _row_spec(NFEAT), _row_spec(1),
                  _full_spec((NFEAT,)), _full_spec((NFEAT, NFEAT))],
        out_specs=_row_spec(NFEAT),
        out_shape=jax.ShapeDtypeStruct((NP, NFEAT), jnp.float32),
    )(agg, hp, s2d, b1, W2)


def _tc3(agg, hp, s2d, b2, W_post, b_post):
    br = 1000
    return pl.pallas_call(
        _tc3_body,
        grid=(N_NODES // br,),
        in_specs=[pl.BlockSpec((2, br, NFEAT), lambda i: (0, i, 0)),
                  pl.BlockSpec((br, NFEAT), lambda i: (i, 0)),
                  pl.BlockSpec((br, 1), lambda i: (i, 0)),
                  _full_spec((NFEAT,)), _full_spec((NFEAT, NCLASS)),
                  _full_spec((NCLASS,))],
        out_specs=pl.BlockSpec((br, NCLASS), lambda i: (i, 0)),
        out_shape=jax.ShapeDtypeStruct((N_NODES, NCLASS), jnp.float32),
    )(agg, hp, s2d, b2, W_post, b_post)


def kernel(x, edge_index, W_pre, b_pre, W1, b1, W2, b2, W_post, b_post):
    row = edge_index[0].astype(jnp.int32)
    col = edge_index[1].astype(jnp.int32)
    npad = EP - N_EDGES
    pad_r = (jnp.arange(npad, dtype=jnp.int32) * 37) % N_NODES
    pad_c = N_NODES + (jnp.arange(npad, dtype=jnp.int32) % 16)
    rowp = jnp.concatenate([row, pad_r])
    colp = jnp.concatenate([col, pad_c])
    rc = jnp.stack([rowp.reshape(NTILES * CHUNKS, K),
                    colp.reshape(NTILES * CHUNKS, K)], axis=1)

    x_p = jnp.pad(x, ((0, NP - N_NODES), (0, 0)))

    dp = _deg_call(rc)
    t1 = _tc1a(x_p, W_pre, b_pre, W1)
    hp1, s2d = _tc1b(t1, dp)
    agg1 = _agg_call(hp1, rc)
    hp2 = _tc2(agg1, hp1, s2d, b1, W2)
    agg2 = _agg_call(hp2, rc)
    return _tc3(agg2, hp2, s2d, b2, W_post, b_post)


# speedup vs baseline: 1.0084x; 1.0084x over previous
import jax
import jax.numpy as jnp
from jax import lax
from jax.experimental import pallas as pl
from jax.experimental.pallas import tpu as pltpu
from jax.experimental.pallas import tpu_sc as plsc

N_NODES = 10000
NP = 10240
NFEAT = 128
NCLASS = 40
N_EDGES = 320000
K = 128
CHUNKS = 80
EPT = K * CHUNKS
NTILES = 32
EP = EPT * NTILES
RPT = NP // 16
BR = 1024


def _mesh():
    return plsc.VectorSubcoreMesh(core_axis_name="c", subcore_axis_name="s")


def _deg_body(rcc_hbm, out_hbm, c0, c1, c2, c3, ones, dz, acc,
              si0, si1, si2, si3, ss0, ss1, ss2, ss3):
    cb = (c0, c1, c2, c3)
    si = (si0, si1, si2, si3)
    ss = (ss0, ss1, ss2, ss3)
    cid = lax.axis_index("c")
    sid = lax.axis_index("s")
    tid = cid * 16 + sid

    for j in range(8):
        ones[pl.ds(j * 16, 16)] = jnp.full((16,), 1.0, jnp.float32)

    def zstep(j, _):
        dz[pl.ds(j * 16, 16)] = jnp.zeros((16,), jnp.float32)
        return 0
    lax.fori_loop(0, RPT // 16, zstep, 0)
    pltpu.sync_copy(dz, acc.at[pl.ds(sid * RPT, RPT)])
    plsc.subcore_barrier()

    cbase = tid * CHUNKS

    def iload(g, j):
        pltpu.async_copy(rcc_hbm.at[cbase + g], cb[j], si[j])

    def iwait(j):
        pltpu.make_async_copy(rcc_hbm.at[cbase], cb[j], si[j]).wait()

    def sstart(j):
        pltpu.async_copy(ones, acc.at[cb[j]], ss[j], add=True)

    def swait(j):
        pltpu.make_async_copy(ones, acc.at[cb[j]], ss[j]).wait()

    iload(0, 0)

    def body(m, _):
        for b in range(4):
            g = 4 * m + b
            if b >= 3:
                swait((b - 3) % 4)
            else:
                @pl.when(m >= 1)
                def _(b=b):
                    swait((b + 1) % 4)
            if b < 3:
                iload(g + 1, b + 1)
            else:
                @pl.when(m < CHUNKS // 4 - 1)
                def _(g=g):
                    iload(g + 1, 0)
            iwait(b)
            sstart(b)
        return 0
    lax.fori_loop(0, CHUNKS // 4, body, 0)
    for j in (1, 2, 3):
        swait(j)

    plsc.subcore_barrier()
    pltpu.sync_copy(acc.at[pl.ds(sid * RPT, RPT)],
                    out_hbm.at[cid, pl.ds(sid * RPT, RPT)])


def _deg_call(rc):
    f = pl.kernel(
        _deg_body,
        out_type=jax.ShapeDtypeStruct((2, NP), jnp.float32),
        mesh=_mesh(),
        scratch_types=[
            pltpu.VMEM((K,), jnp.int32),
            pltpu.VMEM((K,), jnp.int32),
            pltpu.VMEM((K,), jnp.int32),
            pltpu.VMEM((K,), jnp.int32),
            pltpu.VMEM((K,), jnp.float32),
            pltpu.VMEM((RPT,), jnp.float32),
            pltpu.VMEM_SHARED((NP,), jnp.float32),
        ] + [pltpu.SemaphoreType.DMA] * 8,
    )
    return f(rc)


def _agg_body(hp_hbm, rcr_hbm, rcc_hbm, out_hbm,
              rb0, rb1, rb2, rb3, rb4, rb5, rb6, rb7,
              cb0, cb1, cb2, cb3, cb4, cb5, cb6, cb7,
              rows0, rows1, acc,
              si0, si1, si2, si3, si4, si5, si6, si7,
              sg0, sg1, ss0, ss1):
    rb = (rb0, rb1, rb2, rb3, rb4, rb5, rb6, rb7)
    cb = (cb0, cb1, cb2, cb3, cb4, cb5, cb6, cb7)
    rows = (rows0, rows1)
    si = (si0, si1, si2, si3, si4, si5, si6, si7)
    sg = (sg0, sg1)
    ss = (ss0, ss1)
    cid = lax.axis_index("c")
    sid = lax.axis_index("s")
    tid = cid * 16 + sid

    cbase = tid * CHUNKS

    def iload(g, j):
        pltpu.async_copy(rcr_hbm.at[cbase + g], rb[j], si[j])
        pltpu.async_copy(rcc_hbm.at[cbase + g], cb[j], si[j])

    def iwait(j):
        pltpu.make_async_copy(rcr_hbm.at[cbase], rb[j], si[j]).wait()
        pltpu.make_async_copy(rcc_hbm.at[cbase], cb[j], si[j]).wait()

    iload(0, 0)
    iload(1, 1)

    def zstep(r, _):
        for j in range(8):
            rows0[r, pl.ds(j * 16, 16)] = jnp.zeros((16,), jnp.float32)
        return 0
    lax.fori_loop(0, K, zstep, 0)
    for k in range(RPT // K):
        pltpu.async_copy(rows0, acc.at[pl.ds(sid * RPT + k * K, K)], ss0)
    for k in range(RPT // K):
        pltpu.make_async_copy(rows0, acc.at[pl.ds(sid * RPT + k * K, K)],
                              ss0).wait()
    plsc.subcore_barrier()

    def gstart(jc, jr):
        pltpu.async_copy(hp_hbm.at[rb[jc]], rows[jr], sg[jr])

    def gwait(jr):
        pltpu.make_async_copy(hp_hbm.at[rb[0]], rows[jr], sg[jr]).wait()

    def sstart(jc, jr):
        pltpu.async_copy(rows[jr], acc.at[cb[jc]], ss[jr], add=True)

    def swait(jr):
        pltpu.make_async_copy(rows[0], acc.at[cb[0]], ss[jr]).wait()

    def body(m, _):
        for b in range(8):
            g_s = 8 * m + b
            if b >= 2:
                swait(b % 2)
            else:
                @pl.when(m >= 1)
                def _(b=b):
                    swait(b % 2)
            if b < 6:
                iload(g_s + 2, (b + 2) % 8)
            else:
                @pl.when(m < CHUNKS // 8 - 1)
                def _(g_s=g_s, b=b):
                    iload(g_s + 2, (b + 2) % 8)
            iwait(b)
            gstart(b, b % 2)
            if b >= 1:
                gwait((b - 1) % 2)
                sstart((b - 1) % 8, (b - 1) % 2)
            else:
                @pl.when(m >= 1)
                def _():
                    gwait(1)
                    sstart(7, 1)
        return 0
    lax.fori_loop(0, CHUNKS // 8, body, 0)
    gwait(1)
    sstart(7, 1)
    for j in range(2):
        swait(j)

    plsc.subcore_barrier()
    pltpu.sync_copy(acc.at[pl.ds(sid * RPT, RPT)],
                    out_hbm.at[cid, pl.ds(sid * RPT, RPT)])


def _agg_call(hp, rcr, rcc):
    f = pl.kernel(
        _agg_body,
        out_type=jax.ShapeDtypeStruct((2, NP, NFEAT), jnp.float32),
        mesh=_mesh(),
        scratch_types=[pltpu.VMEM((K,), jnp.int32)] * 16
        + [pltpu.VMEM((K, NFEAT), jnp.float32)] * 2
        + [pltpu.VMEM_SHARED((NP, NFEAT), jnp.float32)]
        + [pltpu.SemaphoreType.DMA] * 12,
    )
    return f(hp, rcr, rcc)


def _s_of(dp_ref):
    d = dp_ref[0, :] + dp_ref[1, :] + 1.0
    return lax.rsqrt(d)


def _tc1a_body(x_ref, wpre_ref, bpre_ref, w1_ref, t1_ref):
    h0 = jnp.dot(x_ref[:], wpre_ref[:], preferred_element_type=jnp.float32)
    h0 = h0 + bpre_ref[:][None, :]
    t1_ref[:] = jnp.dot(h0, w1_ref[:], preferred_element_type=jnp.float32)


def _tc1b_body(t1_ref, dp_ref, hp1_ref, s_ref):
    s = _s_of(dp_ref)
    s_ref[:] = s[:, None]
    hp1_ref[:] = t1_ref[:] * s[:, None]


def _tc2_body(agg_ref, hp_ref, s_ref, b_ref, w2_ref, hp2_ref):
    s = s_ref[:]
    pre = s * (agg_ref[0] + agg_ref[1] + hp_ref[:]) + b_ref[:][None, :]
    h1 = jnp.maximum(pre, 0.0)
    t2 = jnp.dot(h1, w2_ref[:], preferred_element_type=jnp.float32)
    hp2_ref[:] = t2 * s


def _tc3_body(agg_ref, hp_ref, s_ref, b_ref, wpost_ref, bpost_ref, out_ref):
    s = s_ref[:]
    pre = s * (agg_ref[0] + agg_ref[1] + hp_ref[:]) + b_ref[:][None, :]
    h2 = jnp.maximum(pre, 0.0)
    logits = jnp.dot(h2, wpost_ref[:], preferred_element_type=jnp.float32)
    logits = logits + bpost_ref[:][None, :]
    m = jnp.max(logits, axis=1, keepdims=True)
    lse = jnp.log(jnp.sum(jnp.exp(logits - m), axis=1, keepdims=True)) + m
    out_ref[:] = logits - lse


def _row_spec(feat):
    return pl.BlockSpec((BR, feat), lambda i: (i, 0))


def _full_spec(shape):
    return pl.BlockSpec(shape, lambda i: tuple(0 for _ in shape))


_dp_spec = pl.BlockSpec((2, BR), lambda i: (0, i))
_agg_spec = pl.BlockSpec((2, BR, NFEAT), lambda i: (0, i, 0))
_grid = (NP // BR,)


def _tc1a(x_p, W_pre, b_pre, W1):
    return pl.pallas_call(
        _tc1a_body,
        grid=_grid,
        in_specs=[_row_spec(NFEAT), _full_spec((NFEAT, NFEAT)),
                  _full_spec((NFEAT,)), _full_spec((NFEAT, NFEAT))],
        out_specs=_row_spec(NFEAT),
        out_shape=jax.ShapeDtypeStruct((NP, NFEAT), jnp.float32),
    )(x_p, W_pre, b_pre, W1)


def _tc1b(t1, dp):
    return pl.pallas_call(
        _tc1b_body,
        grid=_grid,
        in_specs=[_row_spec(NFEAT), _dp_spec],
        out_specs=[_row_spec(NFEAT), _row_spec(1)],
        out_shape=[jax.ShapeDtypeStruct((NP, NFEAT), jnp.float32),
                   jax.ShapeDtypeStruct((NP, 1), jnp.float32)],
    )(t1, dp)


def _tc2(agg, hp, s2d, b1, W2):
    return pl.pallas_call(
        _tc2_body,
        grid=_grid,
        in_specs=[_agg_spec, _row_spec(NFEAT), _row_spec(1),
                  _full_spec((NFEAT,)), _full_spec((NFEAT, NFEAT))],
        out_specs=_row_spec(NFEAT),
        out_shape=jax.ShapeDtypeStruct((NP, NFEAT), jnp.float32),
    )(agg, hp, s2d, b1, W2)


def _tc3(agg, hp, s2d, b2, W_post, b_post):
    br = 1000
    return pl.pallas_call(
        _tc3_body,
        grid=(N_NODES // br,),
        in_specs=[pl.BlockSpec((2, br, NFEAT), lambda i: (0, i, 0)),
                  pl.BlockSpec((br, NFEAT), lambda i: (i, 0)),
                  pl.BlockSpec((br, 1), lambda i: (i, 0)),
                  _full_spec((NFEAT,)), _full_spec((NFEAT, NCLASS)),
                  _full_spec((NCLASS,))],
        out_specs=pl.BlockSpec((br, NCLASS), lambda i: (i, 0)),
        out_shape=jax.ShapeDtypeStruct((N_NODES, NCLASS), jnp.float32),
    )(agg, hp, s2d, b2, W_post, b_post)


def kernel(x, edge_index, W_pre, b_pre, W1, b1, W2, b2, W_post, b_post):
    row = edge_index[0].astype(jnp.int32)
    col = edge_index[1].astype(jnp.int32)
    npad = EP - N_EDGES
    pad_r = (jnp.arange(npad, dtype=jnp.int32) * 37) % N_NODES
    pad_c = N_NODES + (jnp.arange(npad, dtype=jnp.int32) % 16)
    rcr = jnp.concatenate([row, pad_r]).reshape(NTILES * CHUNKS, K)
    rcc = jnp.concatenate([col, pad_c]).reshape(NTILES * CHUNKS, K)

    x_p = jnp.pad(x, ((0, NP - N_NODES), (0, 0)))

    dp = _deg_call(rcc)
    t1 = _tc1a(x_p, W_pre, b_pre, W1)
    hp1, s2d = _tc1b(t1, dp)
    agg1 = _agg_call(hp1, rcr, rcc)
    hp2 = _tc2(agg1, hp1, s2d, b1, W2)
    agg2 = _agg_call(hp2, rcr, rcc)
    return _tc3(agg2, hp2, s2d, b2, W_post, b_post)
